# baseline (speedup 1.0000x reference)
import functools

import jax
import jax.numpy as jnp
from jax import lax
from jax.experimental import pallas as pl
from jax.experimental.pallas import tpu as pltpu
from jax.experimental.pallas import tpu_sc as plsc

N_NODES = 10000
N_EDGES = 320000
D = 128
ED = 16

NC = 2
NS = 16
NW = NC * NS
EP = N_EDGES // NW
K = 80
SEG = 5
EPS = EP // SEG
CH = EPS // K
ES = N_EDGES // SEG
RING = 4
SUB_ROWS = 624
TAIL_ROWS = N_NODES - NS * SUB_ROWS

_mesh = functools.partial(
    plsc.VectorSubcoreMesh, core_axis_name="c", subcore_axis_name="s")


def _pre_body(x_ref, w_ref, a_ref, b_ref):
    ab = jnp.dot(x_ref[...], w_ref[...], preferred_element_type=jnp.float32)
    a_ref[...] = ab[:, :D]
    b_ref[...] = ab[:, D:]


def _precompute(x, w_ab):
    return pl.pallas_call(
        _pre_body,
        out_shape=(jax.ShapeDtypeStruct((N_NODES, D), jnp.float32),
                   jax.ShapeDtypeStruct((N_NODES, D), jnp.float32)),
    )(x, w_ab)


def _gather_body(a_hbm, b_hbm, pidx, ga_hbm, gb_hbm,
                 idxb, ra, rb, sem_a, sem_b, sem_wa, sem_wb):
    c = lax.axis_index("c")
    s = lax.axis_index("s")
    wid = s * NC + c

    pltpu.sync_copy(pidx.at[wid], idxb)

    def gather_descs(j, slot):
        da = pltpu.make_async_copy(a_hbm.at[idxb.at[2 * j, pl.ds(0, K)]],
                                   ra.at[slot], sem_a.at[slot])
        db = pltpu.make_async_copy(b_hbm.at[idxb.at[2 * j + 1, pl.ds(0, K)]],
                                   rb.at[slot], sem_b.at[slot])
        return da, db

    def issue(j, slot):
        da, db = gather_descs(j, slot)
        da.start()
        db.start()

    def drain_writes(slot):
        pltpu.make_async_copy(ra.at[slot], ga_hbm.at[pl.ds(0, K)],
                              sem_wa.at[slot]).wait()
        pltpu.make_async_copy(rb.at[slot], gb_hbm.at[pl.ds(0, K)],
                              sem_wb.at[slot]).wait()

    issue(0, 0)
    issue(1, 1)

    def chunk(j, carry):
        slot = j % RING
        da, db = gather_descs(j, slot)
        da.wait()
        db.wait()
        off = pl.multiple_of(wid * EPS + j * K, 8)
        pltpu.async_copy(ra.at[slot], ga_hbm.at[pl.ds(off, K)],
                         sem_wa.at[slot])
        pltpu.async_copy(rb.at[slot], gb_hbm.at[pl.ds(off, K)],
                         sem_wb.at[slot])

        nxt = (j + 2) % RING

        @pl.when((j + 2 < CH) & (j >= 2))
        def _drain():
            drain_writes(nxt)

        @pl.when(j + 2 < CH)
        def _prefetch():
            issue(j + 2, nxt)

        return carry

    lax.fori_loop(0, CH, chunk, 0, unroll=False)
    for jj in range(CH - RING, CH):
        drain_writes(jj % RING)


def _gather_seg(a, b, pidx):
    return pl.kernel(
        _gather_body,
        out_type=(jax.ShapeDtypeStruct((ES, D), jnp.float32),
                  jax.ShapeDtypeStruct((ES, D), jnp.float32)),
        mesh=_mesh(),
        scratch_types=[
            pltpu.VMEM((2 * CH, 128), jnp.int32),
            pltpu.VMEM((RING, K, D), jnp.float32),
            pltpu.VMEM((RING, K, D), jnp.float32),
            pltpu.SemaphoreType.DMA((RING,)),
            pltpu.SemaphoreType.DMA((RING,)),
            pltpu.SemaphoreType.DMA((RING,)),
            pltpu.SemaphoreType.DMA((RING,)),
        ],
    )(a, b, pidx)


def _edge_mlp_body(ga_ref, gb_ref, ea_ref, w1c_ref, b1_ref, w2_ref, b2_ref,
                   m_ref):
    z = (ga_ref[...] + gb_ref[...]
         + jnp.dot(ea_ref[...], w1c_ref[...], preferred_element_type=jnp.float32)
         + b1_ref[...])
    h = jnp.maximum(z, 0.0).astype(jnp.bfloat16)
    m_ref[...] = (jnp.dot(h, w2_ref[...], preferred_element_type=jnp.float32)
                  + b2_ref[...])


def _edge_mlp(ga, gb, edge_attr, w1c, b1, w2, b2, block_e=4000):
    ne = ga.shape[0]
    grid = ne // block_e
    return pl.pallas_call(
        _edge_mlp_body,
        grid=(grid,),
        in_specs=[
            pl.BlockSpec((block_e, D), lambda i: (i, 0)),
            pl.BlockSpec((block_e, D), lambda i: (i, 0)),
            pl.BlockSpec((block_e, ED), lambda i: (i, 0)),
            pl.BlockSpec((ED, D), lambda i: (0, 0)),
            pl.BlockSpec((1, D), lambda i: (0, 0)),
            pl.BlockSpec((D, D), lambda i: (0, 0)),
            pl.BlockSpec((1, D), lambda i: (0, 0)),
        ],
        out_specs=pl.BlockSpec((block_e, D), lambda i: (i, 0)),
        out_shape=jax.ShapeDtypeStruct((ne, D), jnp.float32),
    )(ga, gb, edge_attr, w1c, b1, w2, b2)


def _scatter_body(m_hbm, idx3d, part_hbm, idx_r, mb, agg, sem_l):
    c = lax.axis_index("c")
    s = lax.axis_index("s")
    wid = s * NC + c

    def zrow(e, carry):
        for v in range(D // 16):
            mb[0, e, pl.ds(v * 16, 16)] = jnp.zeros((16,), jnp.float32)
        return carry

    lax.fori_loop(0, K, zrow, 0, unroll=False)
    for t in range(7):
        zoff = pl.multiple_of(s * SUB_ROWS + t * K, 8)
        pltpu.sync_copy(mb.at[0], agg.at[pl.ds(zoff, K)])
    zoff = pl.multiple_of(s * SUB_ROWS + 7 * K, 8)
    pltpu.sync_copy(mb.at[0, pl.ds(0, 64)], agg.at[pl.ds(zoff, 64)])

    @pl.when(s == NS - 1)
    def _zero_tail():
        pltpu.sync_copy(mb.at[0, pl.ds(0, TAIL_ROWS)],
                        agg.at[pl.ds(NS * SUB_ROWS, TAIL_ROWS)])

    plsc.subcore_barrier()

    pltpu.sync_copy(idx3d.at[wid], idx_r)

    def load(j, slot):
        off = pl.multiple_of(wid * EPS + j * K, 8)
        pltpu.async_copy(m_hbm.at[pl.ds(off, K)], mb.at[slot], sem_l.at[slot])

    load(0, 0)

    def chunk(j, carry):
        slot = j % 2
        pltpu.make_async_copy(m_hbm.at[pl.ds(0, K)], mb.at[slot],
                              sem_l.at[slot]).wait()

        @pl.when(j + 1 < CH)
        def _prefetch():
            load(j + 1, (j + 1) % 2)

        pltpu.sync_copy(mb.at[slot], agg.at[idx_r.at[j]], add=True)
        return carry

    lax.fori_loop(0, CH, chunk, 0, unroll=False)
    plsc.subcore_barrier()

    woff = pl.multiple_of(s * SUB_ROWS, 8)
    pltpu.sync_copy(agg.at[pl.ds(woff, SUB_ROWS)],
                    part_hbm.at[c, pl.ds(woff, SUB_ROWS)])

    @pl.when(s == NS - 1)
    def _write_tail():
        pltpu.sync_copy(agg.at[pl.ds(NS * SUB_ROWS, TAIL_ROWS)],
                        part_hbm.at[c, pl.ds(NS * SUB_ROWS, TAIL_ROWS)])


def _scatter_seg(m, idx3d):
    return pl.kernel(
        _scatter_body,
        out_type=jax.ShapeDtypeStruct((NC, N_NODES, D), jnp.float32),
        mesh=_mesh(),
        scratch_types=[
            pltpu.VMEM((CH, K), jnp.int32),
            pltpu.VMEM((2, K, D), jnp.float32),
            pltpu.VMEM_SHARED((N_NODES, D), jnp.float32),
            pltpu.SemaphoreType.DMA((2,)),
        ],
    )(m, idx3d)


def _node_mlp_body(x_ref, *rest):
    ps, (wnx_ref, wna_ref, bn1_ref, wn2_ref, bn2_ref, o_ref) = (
        rest[:SEG], rest[SEG:])
    agg = ps[0][...].sum(axis=0)
    for p in ps[1:]:
        agg = agg + p[...].sum(axis=0)
    t = (jnp.dot(x_ref[...], wnx_ref[...], preferred_element_type=jnp.float32)
         + jnp.dot(agg, wna_ref[...], preferred_element_type=jnp.float32)
         + bn1_ref[...])
    h = jnp.maximum(t, 0.0)
    o_ref[...] = (jnp.dot(h, wn2_ref[...], preferred_element_type=jnp.float32)
                  + bn2_ref[...])


def _node_mlp(x, parts, wnx, wna, bn1, wn2, bn2, block_n=2000):
    grid = N_NODES // block_n
    pspec = pl.BlockSpec((NC, block_n, D), lambda i: (0, i, 0))
    wspec = pl.BlockSpec((D, D), lambda i: (0, 0))
    bspec = pl.BlockSpec((1, D), lambda i: (0, 0))
    return pl.pallas_call(
        _node_mlp_body,
        grid=(grid,),
        in_specs=[pl.BlockSpec((block_n, D), lambda i: (i, 0))]
        + [pspec] * SEG + [wspec, wspec, bspec, wspec, bspec],
        out_specs=pl.BlockSpec((block_n, D), lambda i: (i, 0)),
        out_shape=jax.ShapeDtypeStruct((N_NODES, D), jnp.float32),
    )(x, *parts, wnx, wna, bn1, wn2, bn2)


def kernel(x, edge_index, edge_attr, W_e1, b_e1, W_e2, b_e2,
           W_n1, b_n1, W_n2, b_n2):
    row = edge_index[0].astype(jnp.int32)
    col = edge_index[1].astype(jnp.int32)

    zpad = jnp.zeros((NW, SEG, CH, 128 - K), jnp.int32)
    r4 = row.reshape(NW, SEG, CH, K)
    c4 = col.reshape(NW, SEG, CH, K)
    rrows = jnp.concatenate([r4, zpad], axis=3)
    crows = jnp.concatenate([c4, zpad], axis=3)
    packed = jnp.stack([rrows, crows], axis=3)
    packed = packed.transpose(1, 0, 2, 3, 4).reshape(SEG, NW, 2 * CH, 128)
    ridx_seg = r4.transpose(1, 0, 2, 3)

    ea_seg = edge_attr.reshape(NW, SEG, EPS, ED).transpose(1, 0, 2, 3)

    w_ab = jnp.concatenate([W_e1[:D], W_e1[D:2 * D]], axis=1)
    a, b = _precompute(x, w_ab)

    w1c = W_e1[2 * D:]
    b1r = b_e1.reshape(1, D)
    w2b = W_e2.astype(jnp.bfloat16)
    b2r = b_e2.reshape(1, D)

    gs = [_gather_seg(a, b, packed[s]) for s in range(SEG)]
    ms = [_edge_mlp(ga, gb, ea_seg[s].reshape(ES, ED), w1c, b1r, w2b, b2r)
          for s, (ga, gb) in enumerate(gs)]
    parts = [_scatter_seg(m, ridx_seg[s]) for s, m in enumerate(ms)]

    out = _node_mlp(x, parts, W_n1[:D], W_n1[D:], b_n1.reshape(1, D),
                    W_n2, b_n2.reshape(1, D))
    return out


# speedup vs baseline: 1.0734x; 1.0734x over previous
import functools

import jax
import jax.numpy as jnp
from jax import lax
from jax.experimental import pallas as pl
from jax.experimental.pallas import tpu as pltpu
from jax.experimental.pallas import tpu_sc as plsc

N_NODES = 10000
N_EDGES = 320000
D = 128
ED = 16

NC = 2
NS = 16
NW = NC * NS
EP = N_EDGES // NW
K = 80
SEG = 1
EPS = EP // SEG
CH = EPS // K
ES = N_EDGES // SEG
RING = 4
SUB_ROWS = 624
TAIL_ROWS = N_NODES - NS * SUB_ROWS

_mesh = functools.partial(
    plsc.VectorSubcoreMesh, core_axis_name="c", subcore_axis_name="s")


def _pre_body(x_ref, w_ref, a_ref, b_ref):
    ab = jnp.dot(x_ref[...], w_ref[...], preferred_element_type=jnp.float32)
    a_ref[...] = ab[:, :D]
    b_ref[...] = ab[:, D:]


def _precompute(x, w_ab):
    return pl.pallas_call(
        _pre_body,
        out_shape=(jax.ShapeDtypeStruct((N_NODES, D), jnp.float32),
                   jax.ShapeDtypeStruct((N_NODES, D), jnp.float32)),
    )(x, w_ab)


def _gather_body(a_hbm, b_hbm, pidx, ga_hbm, gb_hbm,
                 idxb, ra, rb, sem_a, sem_b, sem_wa, sem_wb):
    c = lax.axis_index("c")
    s = lax.axis_index("s")
    wid = s * NC + c

    pltpu.sync_copy(pidx.at[wid], idxb)

    def gather_descs(j, slot):
        da = pltpu.make_async_copy(a_hbm.at[idxb.at[2 * j, pl.ds(0, K)]],
                                   ra.at[slot], sem_a.at[slot])
        db = pltpu.make_async_copy(b_hbm.at[idxb.at[2 * j + 1, pl.ds(0, K)]],
                                   rb.at[slot], sem_b.at[slot])
        return da, db

    def issue(j, slot):
        da, db = gather_descs(j, slot)
        da.start()
        db.start()

    def drain_writes(slot):
        pltpu.make_async_copy(ra.at[slot], ga_hbm.at[pl.ds(0, K)],
                              sem_wa.at[slot]).wait()
        pltpu.make_async_copy(rb.at[slot], gb_hbm.at[pl.ds(0, K)],
                              sem_wb.at[slot]).wait()

    issue(0, 0)
    issue(1, 1)

    def chunk(j, carry):
        slot = j % RING
        da, db = gather_descs(j, slot)
        da.wait()
        db.wait()
        off = pl.multiple_of(wid * EPS + j * K, 8)
        pltpu.async_copy(ra.at[slot], ga_hbm.at[pl.ds(off, K)],
                         sem_wa.at[slot])
        pltpu.async_copy(rb.at[slot], gb_hbm.at[pl.ds(off, K)],
                         sem_wb.at[slot])

        nxt = (j + 2) % RING

        @pl.when((j + 2 < CH) & (j >= 2))
        def _drain():
            drain_writes(nxt)

        @pl.when(j + 2 < CH)
        def _prefetch():
            issue(j + 2, nxt)

        return carry

    lax.fori_loop(0, CH, chunk, 0, unroll=False)
    for jj in range(CH - RING, CH):
        drain_writes(jj % RING)


def _gather_seg(a, b, pidx):
    return pl.kernel(
        _gather_body,
        out_type=(jax.ShapeDtypeStruct((ES, D), jnp.float32),
                  jax.ShapeDtypeStruct((ES, D), jnp.float32)),
        mesh=_mesh(),
        scratch_types=[
            pltpu.VMEM((2 * CH, 128), jnp.int32),
            pltpu.VMEM((RING, K, D), jnp.float32),
            pltpu.VMEM((RING, K, D), jnp.float32),
            pltpu.SemaphoreType.DMA((RING,)),
            pltpu.SemaphoreType.DMA((RING,)),
            pltpu.SemaphoreType.DMA((RING,)),
            pltpu.SemaphoreType.DMA((RING,)),
        ],
    )(a, b, pidx)


def _edge_mlp_body(ga_ref, gb_ref, ea_ref, w1c_ref, b1_ref, w2_ref, b2_ref,
                   m_ref):
    z = (ga_ref[...] + gb_ref[...]
         + jnp.dot(ea_ref[...], w1c_ref[...], preferred_element_type=jnp.float32)
         + b1_ref[...])
    h = jnp.maximum(z, 0.0).astype(jnp.bfloat16)
    m_ref[...] = (jnp.dot(h, w2_ref[...], preferred_element_type=jnp.float32)
                  + b2_ref[...])


def _edge_mlp(ga, gb, edge_attr, w1c, b1, w2, b2, block_e=4000):
    ne = ga.shape[0]
    grid = ne // block_e
    return pl.pallas_call(
        _edge_mlp_body,
        grid=(grid,),
        in_specs=[
            pl.BlockSpec((block_e, D), lambda i: (i, 0)),
            pl.BlockSpec((block_e, D), lambda i: (i, 0)),
            pl.BlockSpec((block_e, ED), lambda i: (i, 0)),
            pl.BlockSpec((ED, D), lambda i: (0, 0)),
            pl.BlockSpec((1, D), lambda i: (0, 0)),
            pl.BlockSpec((D, D), lambda i: (0, 0)),
            pl.BlockSpec((1, D), lambda i: (0, 0)),
        ],
        out_specs=pl.BlockSpec((block_e, D), lambda i: (i, 0)),
        out_shape=jax.ShapeDtypeStruct((ne, D), jnp.float32),
    )(ga, gb, edge_attr, w1c, b1, w2, b2)


def _scatter_body(m_hbm, idx3d, part_hbm, idx_r, mb, agg, sem_l):
    c = lax.axis_index("c")
    s = lax.axis_index("s")
    wid = s * NC + c

    def zrow(e, carry):
        for v in range(D // 16):
            mb[0, e, pl.ds(v * 16, 16)] = jnp.zeros((16,), jnp.float32)
        return carry

    lax.fori_loop(0, K, zrow, 0, unroll=False)
    for t in range(7):
        zoff = pl.multiple_of(s * SUB_ROWS + t * K, 8)
        pltpu.sync_copy(mb.at[0], agg.at[pl.ds(zoff, K)])
    zoff = pl.multiple_of(s * SUB_ROWS + 7 * K, 8)
    pltpu.sync_copy(mb.at[0, pl.ds(0, 64)], agg.at[pl.ds(zoff, 64)])

    @pl.when(s == NS - 1)
    def _zero_tail():
        pltpu.sync_copy(mb.at[0, pl.ds(0, TAIL_ROWS)],
                        agg.at[pl.ds(NS * SUB_ROWS, TAIL_ROWS)])

    plsc.subcore_barrier()

    pltpu.sync_copy(idx3d.at[wid], idx_r)

    def load(j, slot):
        off = pl.multiple_of(wid * EPS + j * K, 8)
        pltpu.async_copy(m_hbm.at[pl.ds(off, K)], mb.at[slot], sem_l.at[slot])

    load(0, 0)

    def chunk(j, carry):
        slot = j % 2
        pltpu.make_async_copy(m_hbm.at[pl.ds(0, K)], mb.at[slot],
                              sem_l.at[slot]).wait()

        @pl.when(j + 1 < CH)
        def _prefetch():
            load(j + 1, (j + 1) % 2)

        pltpu.sync_copy(mb.at[slot], agg.at[idx_r.at[j]], add=True)
        return carry

    lax.fori_loop(0, CH, chunk, 0, unroll=False)
    plsc.subcore_barrier()

    woff = pl.multiple_of(s * SUB_ROWS, 8)
    pltpu.sync_copy(agg.at[pl.ds(woff, SUB_ROWS)],
                    part_hbm.at[c, pl.ds(woff, SUB_ROWS)])

    @pl.when(s == NS - 1)
    def _write_tail():
        pltpu.sync_copy(agg.at[pl.ds(NS * SUB_ROWS, TAIL_ROWS)],
                        part_hbm.at[c, pl.ds(NS * SUB_ROWS, TAIL_ROWS)])


def _scatter_seg(m, idx3d):
    return pl.kernel(
        _scatter_body,
        out_type=jax.ShapeDtypeStruct((NC, N_NODES, D), jnp.float32),
        mesh=_mesh(),
        scratch_types=[
            pltpu.VMEM((CH, K), jnp.int32),
            pltpu.VMEM((2, K, D), jnp.float32),
            pltpu.VMEM_SHARED((N_NODES, D), jnp.float32),
            pltpu.SemaphoreType.DMA((2,)),
        ],
    )(m, idx3d)


def _node_mlp_body(x_ref, *rest):
    ps, (wnx_ref, wna_ref, bn1_ref, wn2_ref, bn2_ref, o_ref) = (
        rest[:SEG], rest[SEG:])
    agg = ps[0][...].sum(axis=0)
    for p in ps[1:]:
        agg = agg + p[...].sum(axis=0)
    t = (jnp.dot(x_ref[...], wnx_ref[...], preferred_element_type=jnp.float32)
         + jnp.dot(agg, wna_ref[...], preferred_element_type=jnp.float32)
         + bn1_ref[...])
    h = jnp.maximum(t, 0.0)
    o_ref[...] = (jnp.dot(h, wn2_ref[...], preferred_element_type=jnp.float32)
                  + bn2_ref[...])


def _node_mlp(x, parts, wnx, wna, bn1, wn2, bn2, block_n=2000):
    grid = N_NODES // block_n
    pspec = pl.BlockSpec((NC, block_n, D), lambda i: (0, i, 0))
    wspec = pl.BlockSpec((D, D), lambda i: (0, 0))
    bspec = pl.BlockSpec((1, D), lambda i: (0, 0))
    return pl.pallas_call(
        _node_mlp_body,
        grid=(grid,),
        in_specs=[pl.BlockSpec((block_n, D), lambda i: (i, 0))]
        + [pspec] * SEG + [wspec, wspec, bspec, wspec, bspec],
        out_specs=pl.BlockSpec((block_n, D), lambda i: (i, 0)),
        out_shape=jax.ShapeDtypeStruct((N_NODES, D), jnp.float32),
    )(x, *parts, wnx, wna, bn1, wn2, bn2)


def kernel(x, edge_index, edge_attr, W_e1, b_e1, W_e2, b_e2,
           W_n1, b_n1, W_n2, b_n2):
    row = edge_index[0].astype(jnp.int32)
    col = edge_index[1].astype(jnp.int32)

    zpad = jnp.zeros((NW, SEG, CH, 128 - K), jnp.int32)
    r4 = row.reshape(NW, SEG, CH, K)
    c4 = col.reshape(NW, SEG, CH, K)
    rrows = jnp.concatenate([r4, zpad], axis=3)
    crows = jnp.concatenate([c4, zpad], axis=3)
    packed = jnp.stack([rrows, crows], axis=3)
    packed = packed.transpose(1, 0, 2, 3, 4).reshape(SEG, NW, 2 * CH, 128)
    ridx_seg = r4.transpose(1, 0, 2, 3)

    ea_seg = edge_attr.reshape(NW, SEG, EPS, ED).transpose(1, 0, 2, 3)

    w_ab = jnp.concatenate([W_e1[:D], W_e1[D:2 * D]], axis=1)
    a, b = _precompute(x, w_ab)

    w1c = W_e1[2 * D:]
    b1r = b_e1.reshape(1, D)
    w2b = W_e2.astype(jnp.bfloat16)
    b2r = b_e2.reshape(1, D)

    gs = [_gather_seg(a, b, packed[s]) for s in range(SEG)]
    ms = [_edge_mlp(ga, gb, ea_seg[s].reshape(ES, ED), w1c, b1r, w2b, b2r)
          for s, (ga, gb) in enumerate(gs)]
    parts = [_scatter_seg(m, ridx_seg[s]) for s, m in enumerate(ms)]

    out = _node_mlp(x, parts, W_n1[:D], W_n1[D:], b_n1.reshape(1, D),
                    W_n2, b_n2.reshape(1, D))
    return out


# speedup vs baseline: 1.0772x; 1.0036x over previous
import functools

import jax
import jax.numpy as jnp
from jax import lax
from jax.experimental import pallas as pl
from jax.experimental.pallas import tpu as pltpu
from jax.experimental.pallas import tpu_sc as plsc

N_NODES = 10000
N_EDGES = 320000
D = 128
DW = D // 2
ED = 16

NC = 2
NS = 16
NW = NC * NS
EP = N_EDGES // NW
K = 80
SEG = 1
EPS = EP // SEG
CH = EPS // K
ES = N_EDGES // SEG
RING = 4
SUB_ROWS = 624
TAIL_ROWS = N_NODES - NS * SUB_ROWS

_mesh = functools.partial(
    plsc.VectorSubcoreMesh, core_axis_name="c", subcore_axis_name="s")


def _pre_body(x_ref, w_ref, a_ref, b_ref):
    ab = jnp.dot(x_ref[...], w_ref[...], preferred_element_type=jnp.float32)
    a_ref[...] = ab[:, :D]
    b_ref[...] = ab[:, D:]


def _precompute(x, w_ab):
    return pl.pallas_call(
        _pre_body,
        out_shape=(jax.ShapeDtypeStruct((N_NODES, D), jnp.float32),
                   jax.ShapeDtypeStruct((N_NODES, D), jnp.float32)),
    )(x, w_ab)


def _gather_body(a_hbm, b_hbm, pidx, ga_hbm, gb_hbm,
                 idxb, ra, rb, sem_a, sem_b, sem_wa, sem_wb):
    c = lax.axis_index("c")
    s = lax.axis_index("s")
    wid = s * NC + c

    pltpu.sync_copy(pidx.at[wid], idxb)

    def gather_descs(j, slot):
        da = pltpu.make_async_copy(a_hbm.at[idxb.at[2 * j, pl.ds(0, K)]],
                                   ra.at[slot], sem_a.at[slot])
        db = pltpu.make_async_copy(b_hbm.at[idxb.at[2 * j + 1, pl.ds(0, K)]],
                                   rb.at[slot], sem_b.at[slot])
        return da, db

    def issue(j, slot):
        da, db = gather_descs(j, slot)
        da.start()
        db.start()

    def drain_writes(slot):
        pltpu.make_async_copy(ra.at[slot], ga_hbm.at[pl.ds(0, K)],
                              sem_wa.at[slot]).wait()
        pltpu.make_async_copy(rb.at[slot], gb_hbm.at[pl.ds(0, K)],
                              sem_wb.at[slot]).wait()

    issue(0, 0)
    issue(1, 1)

    def chunk(j, carry):
        slot = j % RING
        da, db = gather_descs(j, slot)
        da.wait()
        db.wait()
        off = pl.multiple_of(wid * EPS + j * K, 8)
        pltpu.async_copy(ra.at[slot], ga_hbm.at[pl.ds(off, K)],
                         sem_wa.at[slot])
        pltpu.async_copy(rb.at[slot], gb_hbm.at[pl.ds(off, K)],
                         sem_wb.at[slot])

        nxt = (j + 2) % RING

        @pl.when((j + 2 < CH) & (j >= 2))
        def _drain():
            drain_writes(nxt)

        @pl.when(j + 2 < CH)
        def _prefetch():
            issue(j + 2, nxt)

        return carry

    lax.fori_loop(0, CH, chunk, 0, unroll=False)
    for jj in range(CH - RING, CH):
        drain_writes(jj % RING)


def _gather_seg(a, b, pidx):
    return pl.kernel(
        _gather_body,
        out_type=(jax.ShapeDtypeStruct((ES, D), jnp.float32),
                  jax.ShapeDtypeStruct((ES, D), jnp.float32)),
        mesh=_mesh(),
        scratch_types=[
            pltpu.VMEM((2 * CH, 128), jnp.int32),
            pltpu.VMEM((RING, K, D), jnp.float32),
            pltpu.VMEM((RING, K, D), jnp.float32),
            pltpu.SemaphoreType.DMA((RING,)),
            pltpu.SemaphoreType.DMA((RING,)),
            pltpu.SemaphoreType.DMA((RING,)),
            pltpu.SemaphoreType.DMA((RING,)),
        ],
    )(a, b, pidx)


def _edge_mlp_body(ga_ref, gb_ref, ea_ref, w1c_ref, b1_ref, w2_ref, b2_ref,
                   m_ref):
    z = (ga_ref[...] + gb_ref[...]
         + jnp.dot(ea_ref[...], w1c_ref[...], preferred_element_type=jnp.float32)
         + b1_ref[...])
    h = jnp.maximum(z, 0.0).astype(jnp.bfloat16)
    m_ref[...] = (jnp.dot(h, w2_ref[...], preferred_element_type=jnp.float32)
                  + b2_ref[...])


def _edge_mlp(ga, gb, edge_attr, w1c, b1, w2, b2, block_e=8000):
    ne = ga.shape[0]
    grid = ne // block_e
    return pl.pallas_call(
        _edge_mlp_body,
        grid=(grid,),
        in_specs=[
            pl.BlockSpec((block_e, D), lambda i: (i, 0)),
            pl.BlockSpec((block_e, D), lambda i: (i, 0)),
            pl.BlockSpec((block_e, ED), lambda i: (i, 0)),
            pl.BlockSpec((ED, D), lambda i: (0, 0)),
            pl.BlockSpec((1, D), lambda i: (0, 0)),
            pl.BlockSpec((D, D), lambda i: (0, 0)),
            pl.BlockSpec((1, D), lambda i: (0, 0)),
        ],
        out_specs=pl.BlockSpec((block_e, D), lambda i: (i, 0)),
        out_shape=jax.ShapeDtypeStruct((ne, D), jnp.float32),
    )(ga, gb, edge_attr, w1c, b1, w2, b2)


def _scatter_body(m_hbm, idx3d, part_hbm, idx_r, mb, agg, sem_l):
    c = lax.axis_index("c")
    s = lax.axis_index("s")
    wid = s * NC + c

    def zrow(e, carry):
        for v in range(D // 16):
            mb[0, e, pl.ds(v * 16, 16)] = jnp.zeros((16,), jnp.float32)
        return carry

    lax.fori_loop(0, K, zrow, 0, unroll=False)
    for t in range(7):
        zoff = pl.multiple_of(s * SUB_ROWS + t * K, 8)
        pltpu.sync_copy(mb.at[0], agg.at[pl.ds(zoff, K)])
    zoff = pl.multiple_of(s * SUB_ROWS + 7 * K, 8)
    pltpu.sync_copy(mb.at[0, pl.ds(0, 64)], agg.at[pl.ds(zoff, 64)])

    @pl.when(s == NS - 1)
    def _zero_tail():
        pltpu.sync_copy(mb.at[0, pl.ds(0, TAIL_ROWS)],
                        agg.at[pl.ds(NS * SUB_ROWS, TAIL_ROWS)])

    plsc.subcore_barrier()

    pltpu.sync_copy(idx3d.at[wid], idx_r)

    def load(j, slot):
        off = pl.multiple_of(wid * EPS + j * K, 8)
        pltpu.async_copy(m_hbm.at[pl.ds(off, K)], mb.at[slot], sem_l.at[slot])

    load(0, 0)

    def chunk(j, carry):
        slot = j % 2
        pltpu.make_async_copy(m_hbm.at[pl.ds(0, K)], mb.at[slot],
                              sem_l.at[slot]).wait()

        @pl.when(j + 1 < CH)
        def _prefetch():
            load(j + 1, (j + 1) % 2)

        pltpu.sync_copy(mb.at[slot], agg.at[idx_r.at[j]], add=True)
        return carry

    lax.fori_loop(0, CH, chunk, 0, unroll=False)
    plsc.subcore_barrier()

    woff = pl.multiple_of(s * SUB_ROWS, 8)
    pltpu.sync_copy(agg.at[pl.ds(woff, SUB_ROWS)],
                    part_hbm.at[c, pl.ds(woff, SUB_ROWS)])

    @pl.when(s == NS - 1)
    def _write_tail():
        pltpu.sync_copy(agg.at[pl.ds(NS * SUB_ROWS, TAIL_ROWS)],
                        part_hbm.at[c, pl.ds(NS * SUB_ROWS, TAIL_ROWS)])


def _scatter_seg(m, idx3d):
    return pl.kernel(
        _scatter_body,
        out_type=jax.ShapeDtypeStruct((NC, N_NODES, D), jnp.float32),
        mesh=_mesh(),
        scratch_types=[
            pltpu.VMEM((CH, K), jnp.int32),
            pltpu.VMEM((2, K, D), jnp.float32),
            pltpu.VMEM_SHARED((N_NODES, D), jnp.float32),
            pltpu.SemaphoreType.DMA((2,)),
        ],
    )(m, idx3d)


def _node_mlp_body(x_ref, *rest):
    ps, (wnx_ref, wna_ref, bn1_ref, wn2_ref, bn2_ref, o_ref) = (
        rest[:SEG], rest[SEG:])
    agg = ps[0][...].sum(axis=0)
    for p in ps[1:]:
        agg = agg + p[...].sum(axis=0)
    t = (jnp.dot(x_ref[...], wnx_ref[...], preferred_element_type=jnp.float32)
         + jnp.dot(agg, wna_ref[...], preferred_element_type=jnp.float32)
         + bn1_ref[...])
    h = jnp.maximum(t, 0.0)
    o_ref[...] = (jnp.dot(h, wn2_ref[...], preferred_element_type=jnp.float32)
                  + bn2_ref[...])


def _node_mlp(x, parts, wnx, wna, bn1, wn2, bn2, block_n=2000):
    grid = N_NODES // block_n
    pspec = pl.BlockSpec((NC, block_n, D), lambda i: (0, i, 0))
    wspec = pl.BlockSpec((D, D), lambda i: (0, 0))
    bspec = pl.BlockSpec((1, D), lambda i: (0, 0))
    return pl.pallas_call(
        _node_mlp_body,
        grid=(grid,),
        in_specs=[pl.BlockSpec((block_n, D), lambda i: (i, 0))]
        + [pspec] * SEG + [wspec, wspec, bspec, wspec, bspec],
        out_specs=pl.BlockSpec((block_n, D), lambda i: (i, 0)),
        out_shape=jax.ShapeDtypeStruct((N_NODES, D), jnp.float32),
    )(x, *parts, wnx, wna, bn1, wn2, bn2)


def kernel(x, edge_index, edge_attr, W_e1, b_e1, W_e2, b_e2,
           W_n1, b_n1, W_n2, b_n2):
    row = edge_index[0].astype(jnp.int32)
    col = edge_index[1].astype(jnp.int32)

    zpad = jnp.zeros((NW, SEG, CH, 128 - K), jnp.int32)
    r4 = row.reshape(NW, SEG, CH, K)
    c4 = col.reshape(NW, SEG, CH, K)
    rrows = jnp.concatenate([r4, zpad], axis=3)
    crows = jnp.concatenate([c4, zpad], axis=3)
    packed = jnp.stack([rrows, crows], axis=3)
    packed = packed.transpose(1, 0, 2, 3, 4).reshape(SEG, NW, 2 * CH, 128)
    ridx_seg = r4.transpose(1, 0, 2, 3)

    ea_seg = edge_attr.reshape(NW, SEG, EPS, ED).transpose(1, 0, 2, 3)

    w_ab = jnp.concatenate([W_e1[:D], W_e1[D:2 * D]], axis=1)
    a, b = _precompute(x, w_ab)

    w1c = W_e1[2 * D:]
    b1r = b_e1.reshape(1, D)
    w2b = W_e2.astype(jnp.bfloat16)
    b2r = b_e2.reshape(1, D)

    gs = [_gather_seg(a, b, packed[s]) for s in range(SEG)]
    ms = [_edge_mlp(ga, gb, ea_seg[s].reshape(ES, ED), w1c, b1r, w2b, b2r)
          for s, (ga, gb) in enumerate(gs)]
    parts = [_scatter_seg(m, ridx_seg[s]) for s, m in enumerate(ms)]

    out = _node_mlp(x, parts, W_n1[:D], W_n1[D:], b_n1.reshape(1, D),
                    W_n2, b_n2.reshape(1, D))
    return out


# speedup vs baseline: 1.0781x; 1.0008x over previous
import functools

import jax
import jax.numpy as jnp
from jax import lax
from jax.experimental import pallas as pl
from jax.experimental.pallas import tpu as pltpu
from jax.experimental.pallas import tpu_sc as plsc

N_NODES = 10000
N_EDGES = 320000
D = 128
DW = D // 2
ED = 16

NC = 2
NS = 16
NW = NC * NS
EP = N_EDGES // NW
K = 80
SEG = 1
EPS = EP // SEG
CH = EPS // K
ES = N_EDGES // SEG
RING = 4
SUB_ROWS = 624
TAIL_ROWS = N_NODES - NS * SUB_ROWS

_mesh = functools.partial(
    plsc.VectorSubcoreMesh, core_axis_name="c", subcore_axis_name="s")


def _pre_body(x_ref, w_ref, a_ref, b_ref):
    ab = jnp.dot(x_ref[...], w_ref[...], preferred_element_type=jnp.float32)
    a_ref[...] = ab[:, :D]
    b_ref[...] = ab[:, D:]


def _precompute(x, w_ab):
    return pl.pallas_call(
        _pre_body,
        out_shape=(jax.ShapeDtypeStruct((N_NODES, D), jnp.float32),
                   jax.ShapeDtypeStruct((N_NODES, D), jnp.float32)),
    )(x, w_ab)


def _gather_body(a_hbm, b_hbm, pidx, ga_hbm, gb_hbm,
                 idxb, ra, rb, sem_a, sem_b, sem_wa, sem_wb):
    c = lax.axis_index("c")
    s = lax.axis_index("s")
    wid = s * NC + c

    pltpu.sync_copy(pidx.at[wid], idxb)

    def gather_descs(j, slot):
        da = pltpu.make_async_copy(a_hbm.at[idxb.at[2 * j, pl.ds(0, K)]],
                                   ra.at[slot], sem_a.at[slot])
        db = pltpu.make_async_copy(b_hbm.at[idxb.at[2 * j + 1, pl.ds(0, K)]],
                                   rb.at[slot], sem_b.at[slot])
        return da, db

    def issue(j, slot):
        da, db = gather_descs(j, slot)
        da.start()
        db.start()

    def drain_writes(slot):
        pltpu.make_async_copy(ra.at[slot], ga_hbm.at[pl.ds(0, K)],
                              sem_wa.at[slot]).wait()
        pltpu.make_async_copy(rb.at[slot], gb_hbm.at[pl.ds(0, K)],
                              sem_wb.at[slot]).wait()

    issue(0, 0)
    issue(1, 1)
    issue(2, 2)

    def chunk(j, carry):
        slot = j % RING
        da, db = gather_descs(j, slot)
        da.wait()
        db.wait()
        off = pl.multiple_of(wid * EPS + j * K, 8)
        pltpu.async_copy(ra.at[slot], ga_hbm.at[pl.ds(off, K)],
                         sem_wa.at[slot])
        pltpu.async_copy(rb.at[slot], gb_hbm.at[pl.ds(off, K)],
                         sem_wb.at[slot])

        nxt = (j + 3) % RING

        @pl.when((j + 3 < CH) & (j >= 1))
        def _drain():
            drain_writes(nxt)

        @pl.when(j + 3 < CH)
        def _prefetch():
            issue(j + 3, nxt)

        return carry

    lax.fori_loop(0, CH, chunk, 0, unroll=False)
    for jj in range(CH - RING, CH):
        drain_writes(jj % RING)


def _gather_seg(a, b, pidx):
    return pl.kernel(
        _gather_body,
        out_type=(jax.ShapeDtypeStruct((ES, D), jnp.float32),
                  jax.ShapeDtypeStruct((ES, D), jnp.float32)),
        mesh=_mesh(),
        scratch_types=[
            pltpu.VMEM((2 * CH, 128), jnp.int32),
            pltpu.VMEM((RING, K, D), jnp.float32),
            pltpu.VMEM((RING, K, D), jnp.float32),
            pltpu.SemaphoreType.DMA((RING,)),
            pltpu.SemaphoreType.DMA((RING,)),
            pltpu.SemaphoreType.DMA((RING,)),
            pltpu.SemaphoreType.DMA((RING,)),
        ],
    )(a, b, pidx)


def _edge_mlp_body(ga_ref, gb_ref, ea_ref, w1c_ref, b1_ref, w2_ref, b2_ref,
                   m_ref):
    z = (ga_ref[...] + gb_ref[...]
         + jnp.dot(ea_ref[...], w1c_ref[...], preferred_element_type=jnp.float32)
         + b1_ref[...])
    h = jnp.maximum(z, 0.0).astype(jnp.bfloat16)
    m_ref[...] = (jnp.dot(h, w2_ref[...], preferred_element_type=jnp.float32)
                  + b2_ref[...])


def _edge_mlp(ga, gb, edge_attr, w1c, b1, w2, b2, block_e=8000):
    ne = ga.shape[0]
    grid = ne // block_e
    return pl.pallas_call(
        _edge_mlp_body,
        grid=(grid,),
        in_specs=[
            pl.BlockSpec((block_e, D), lambda i: (i, 0)),
            pl.BlockSpec((block_e, D), lambda i: (i, 0)),
            pl.BlockSpec((block_e, ED), lambda i: (i, 0)),
            pl.BlockSpec((ED, D), lambda i: (0, 0)),
            pl.BlockSpec((1, D), lambda i: (0, 0)),
            pl.BlockSpec((D, D), lambda i: (0, 0)),
            pl.BlockSpec((1, D), lambda i: (0, 0)),
        ],
        out_specs=pl.BlockSpec((block_e, D), lambda i: (i, 0)),
        out_shape=jax.ShapeDtypeStruct((ne, D), jnp.float32),
    )(ga, gb, edge_attr, w1c, b1, w2, b2)


def _scatter_body(m_hbm, idx3d, part_hbm, idx_r, mb, agg, sem_l):
    c = lax.axis_index("c")
    s = lax.axis_index("s")
    wid = s * NC + c

    def zrow(e, carry):
        for v in range(D // 16):
            mb[0, e, pl.ds(v * 16, 16)] = jnp.zeros((16,), jnp.float32)
        return carry

    lax.fori_loop(0, K, zrow, 0, unroll=False)
    for t in range(7):
        zoff = pl.multiple_of(s * SUB_ROWS + t * K, 8)
        pltpu.sync_copy(mb.at[0], agg.at[pl.ds(zoff, K)])
    zoff = pl.multiple_of(s * SUB_ROWS + 7 * K, 8)
    pltpu.sync_copy(mb.at[0, pl.ds(0, 64)], agg.at[pl.ds(zoff, 64)])

    @pl.when(s == NS - 1)
    def _zero_tail():
        pltpu.sync_copy(mb.at[0, pl.ds(0, TAIL_ROWS)],
                        agg.at[pl.ds(NS * SUB_ROWS, TAIL_ROWS)])

    plsc.subcore_barrier()

    pltpu.sync_copy(idx3d.at[wid], idx_r)

    def load(j, slot):
        off = pl.multiple_of(wid * EPS + j * K, 8)
        pltpu.async_copy(m_hbm.at[pl.ds(off, K)], mb.at[slot], sem_l.at[slot])

    load(0, 0)

    def chunk(j, carry):
        slot = j % 2
        pltpu.make_async_copy(m_hbm.at[pl.ds(0, K)], mb.at[slot],
                              sem_l.at[slot]).wait()

        @pl.when(j + 1 < CH)
        def _prefetch():
            load(j + 1, (j + 1) % 2)

        pltpu.sync_copy(mb.at[slot], agg.at[idx_r.at[j]], add=True)
        return carry

    lax.fori_loop(0, CH, chunk, 0, unroll=False)
    plsc.subcore_barrier()

    woff = pl.multiple_of(s * SUB_ROWS, 8)
    pltpu.sync_copy(agg.at[pl.ds(woff, SUB_ROWS)],
                    part_hbm.at[c, pl.ds(woff, SUB_ROWS)])

    @pl.when(s == NS - 1)
    def _write_tail():
        pltpu.sync_copy(agg.at[pl.ds(NS * SUB_ROWS, TAIL_ROWS)],
                        part_hbm.at[c, pl.ds(NS * SUB_ROWS, TAIL_ROWS)])


def _scatter_seg(m, idx3d):
    return pl.kernel(
        _scatter_body,
        out_type=jax.ShapeDtypeStruct((NC, N_NODES, D), jnp.float32),
        mesh=_mesh(),
        scratch_types=[
            pltpu.VMEM((CH, K), jnp.int32),
            pltpu.VMEM((2, K, D), jnp.float32),
            pltpu.VMEM_SHARED((N_NODES, D), jnp.float32),
            pltpu.SemaphoreType.DMA((2,)),
        ],
    )(m, idx3d)


def _node_mlp_body(x_ref, *rest):
    ps, (wnx_ref, wna_ref, bn1_ref, wn2_ref, bn2_ref, o_ref) = (
        rest[:SEG], rest[SEG:])
    agg = ps[0][...].sum(axis=0)
    for p in ps[1:]:
        agg = agg + p[...].sum(axis=0)
    t = (jnp.dot(x_ref[...], wnx_ref[...], preferred_element_type=jnp.float32)
         + jnp.dot(agg, wna_ref[...], preferred_element_type=jnp.float32)
         + bn1_ref[...])
    h = jnp.maximum(t, 0.0)
    o_ref[...] = (jnp.dot(h, wn2_ref[...], preferred_element_type=jnp.float32)
                  + bn2_ref[...])


def _node_mlp(x, parts, wnx, wna, bn1, wn2, bn2, block_n=2000):
    grid = N_NODES // block_n
    pspec = pl.BlockSpec((NC, block_n, D), lambda i: (0, i, 0))
    wspec = pl.BlockSpec((D, D), lambda i: (0, 0))
    bspec = pl.BlockSpec((1, D), lambda i: (0, 0))
    return pl.pallas_call(
        _node_mlp_body,
        grid=(grid,),
        in_specs=[pl.BlockSpec((block_n, D), lambda i: (i, 0))]
        + [pspec] * SEG + [wspec, wspec, bspec, wspec, bspec],
        out_specs=pl.BlockSpec((block_n, D), lambda i: (i, 0)),
        out_shape=jax.ShapeDtypeStruct((N_NODES, D), jnp.float32),
    )(x, *parts, wnx, wna, bn1, wn2, bn2)


def kernel(x, edge_index, edge_attr, W_e1, b_e1, W_e2, b_e2,
           W_n1, b_n1, W_n2, b_n2):
    row = edge_index[0].astype(jnp.int32)
    col = edge_index[1].astype(jnp.int32)

    zpad = jnp.zeros((NW, SEG, CH, 128 - K), jnp.int32)
    r4 = row.reshape(NW, SEG, CH, K)
    c4 = col.reshape(NW, SEG, CH, K)
    rrows = jnp.concatenate([r4, zpad], axis=3)
    crows = jnp.concatenate([c4, zpad], axis=3)
    packed = jnp.stack([rrows, crows], axis=3)
    packed = packed.transpose(1, 0, 2, 3, 4).reshape(SEG, NW, 2 * CH, 128)
    ridx_seg = r4.transpose(1, 0, 2, 3)

    ea_seg = edge_attr.reshape(NW, SEG, EPS, ED).transpose(1, 0, 2, 3)

    w_ab = jnp.concatenate([W_e1[:D], W_e1[D:2 * D]], axis=1)
    a, b = _precompute(x, w_ab)

    w1c = W_e1[2 * D:]
    b1r = b_e1.reshape(1, D)
    w2b = W_e2.astype(jnp.bfloat16)
    b2r = b_e2.reshape(1, D)

    gs = [_gather_seg(a, b, packed[s]) for s in range(SEG)]
    ms = [_edge_mlp(ga, gb, ea_seg[s].reshape(ES, ED), w1c, b1r, w2b, b2r)
          for s, (ga, gb) in enumerate(gs)]
    parts = [_scatter_seg(m, ridx_seg[s]) for s, m in enumerate(ms)]

    out = _node_mlp(x, parts, W_n1[:D], W_n1[D:], b_n1.reshape(1, D),
                    W_n2, b_n2.reshape(1, D))
    return out
